# Initial kernel scaffold; baseline (speedup 1.0000x reference)
import jax
import jax.numpy as jnp
from jax.experimental import pallas as pl


def kernel(input, adj, weight1, weight2, bias):
    raise NotImplementedError("write your pallas kernel here")


# speedup vs baseline: 1.0684x; 1.0684x over previous
import jax
import jax.numpy as jnp
from jax.experimental import pallas as pl
from jax.experimental.pallas import tpu as pltpu

N = 10000
F = 128
BN = 400


def _body(inp_ref, adj_ref, w1_ref, w2_ref, b_ref, out_ref, support_ref):
    i = pl.program_id(0)

    @pl.when(i == 0)
    def _():
        support_ref[...] = jnp.dot(
            inp_ref[...], w2_ref[...], preferred_element_type=jnp.float32
        )

    agg = jnp.dot(adj_ref[...], support_ref[...], preferred_element_type=jnp.float32)
    loc = jnp.dot(
        inp_ref[pl.ds(i * BN, BN), :], w1_ref[...],
        preferred_element_type=jnp.float32,
    )
    out_ref[...] = agg + loc + b_ref[...]


def kernel(input, adj, weight1, weight2, bias):
    inp2d = input.reshape(N, F)
    bias2d = bias.reshape(1, F)
    out = pl.pallas_call(
        _body,
        grid=(N // BN,),
        in_specs=[
            pl.BlockSpec((N, F), lambda i: (0, 0)),
            pl.BlockSpec((BN, N), lambda i: (i, 0)),
            pl.BlockSpec((F, F), lambda i: (0, 0)),
            pl.BlockSpec((F, F), lambda i: (0, 0)),
            pl.BlockSpec((1, F), lambda i: (0, 0)),
        ],
        out_specs=pl.BlockSpec((BN, F), lambda i: (i, 0)),
        out_shape=jax.ShapeDtypeStruct((N, F), jnp.float32),
        scratch_shapes=[pltpu.VMEM((N, F), jnp.float32)],
    )(inp2d, adj, weight1, weight2, bias2d)
    return out.reshape(1, N, F)
